# baseline (speedup 1.0000x reference)
import numpy as np
import jax
import jax.numpy as jnp
from jax.experimental import pallas as pl
from jax.experimental.pallas import tpu as pltpu

_OUT = 64
_TOTAL_C = 1440
_B = 4


def _pool_matrix() -> np.ndarray:
    m = np.zeros((128, _OUT), dtype=np.float32)
    idx = np.arange(_OUT)
    m[2 * idx, idx] = 0.25
    m[2 * idx + 1, idx] = 0.25
    return m


_RXT0 = _pool_matrix()


def _pool_body(x_ref, rxt_ref, o_ref):
    s = (x_ref[0:1, :, 0::2, :] + x_ref[0:1, :, 1::2, :])[0]
    y = jnp.dot(s.reshape(96 * _OUT, 128), rxt_ref[...],
                preferred_element_type=jnp.float32)
    o_ref[0] = y.reshape(96, _OUT, _OUT)


def _pool_feat0(feat0):
    return pl.pallas_call(
        _pool_body,
        grid=(_B,),
        in_specs=[pl.BlockSpec((1, 96, 128, 128), lambda b: (b, 0, 0, 0)),
                  pl.BlockSpec((128, _OUT), lambda b: (0, 0))],
        out_specs=pl.BlockSpec((1, 96, _OUT, _OUT), lambda b: (b, 0, 0, 0)),
        out_shape=jax.ShapeDtypeStruct((_B, 96, _OUT, _OUT), jnp.float32),
    )(feat0, _RXT0)


def _upsample_half(x, xd, xu, scale, weights, ch_base, o_ref, scr_ref):
    half = scale // 2
    c = x.shape[-1]
    for k in range(c // 128):
        lo = 128 * k
        xc, xdc, xuc = x[:, :, lo:lo + 128], xd[:, :, lo:lo + 128], xu[:, :, lo:lo + 128]
        rows = [w * xdc + (1.0 - w) * xc for w in weights[:half]]
        rows += [w * xuc + (1.0 - w) * xc for w in weights[half:]]
        for p, r in enumerate(rows):
            rd = jnp.concatenate([r[:, :1], r[:, :-1]], axis=1)
            ru = jnp.concatenate([r[:, 1:], r[:, -1:]], axis=1)
            cols = [w * rd + (1.0 - w) * r for w in weights[:half]]
            cols += [w * ru + (1.0 - w) * r for w in weights[half:]]
            for q, v in enumerate(cols):
                if scale == 2:
                    scr_ref[p::2, q::2, :] = v
                else:
                    scr_ref[p::4, q::4, :] = v
        o_ref[0:1, :, :, ch_base + lo:ch_base + lo + 128] = scr_ref[...][None]


_NH = 4
_HB = _OUT // _NH


def _body(p0_ref, f1_ref, f2_ref, f3_ref, o_ref, scr_ref):
    h = pl.program_id(1)

    o_ref[0:1, :, :, 0:96] = p0_ref[0:1]
    o_ref[0:1, :, :, 96:288] = f1_ref[0:1]

    def level(x_ref, scale, weights, ch_base):
        n = x_ref.shape[1]
        m = n // _NH

        for hv in range(_NH):
            @pl.when(h == hv)
            def _(hv=hv):
                x = x_ref[0, hv * m:(hv + 1) * m]
                if hv == 0:
                    xd = jnp.concatenate([x[:1], x[:-1]], axis=0)
                else:
                    xd = x_ref[0, hv * m - 1:(hv + 1) * m - 1]
                if hv == _NH - 1:
                    xu = jnp.concatenate([x[1:], x[-1:]], axis=0)
                else:
                    xu = x_ref[0, hv * m + 1:(hv + 1) * m + 1]
                _upsample_half(x, xd, xu, scale, weights, ch_base, o_ref, scr_ref)

    level(f2_ref, 2, (0.25, 0.25), 288)
    level(f3_ref, 4, (0.375, 0.125, 0.125, 0.375), 672)


def kernel(feat0, feat1, feat2, feat3):
    p0 = jnp.transpose(_pool_feat0(feat0), (0, 2, 3, 1))
    f1 = jnp.transpose(feat1, (0, 2, 3, 1))
    f2 = jnp.transpose(feat2, (0, 2, 3, 1))
    f3 = jnp.transpose(feat3, (0, 2, 3, 1))
    out = pl.pallas_call(
        _body,
        grid=(_B, _NH),
        in_specs=[pl.BlockSpec((1, _HB, _OUT, 96), lambda b, h: (b, h, 0, 0)),
                  pl.BlockSpec((1, _HB, _OUT, 192), lambda b, h: (b, h, 0, 0)),
                  pl.BlockSpec((1, 32, 32, 384), lambda b, h: (b, 0, 0, 0)),
                  pl.BlockSpec((1, 16, 16, 768), lambda b, h: (b, 0, 0, 0))],
        out_specs=pl.BlockSpec((1, _HB, _OUT, _TOTAL_C),
                               lambda b, h: (b, h, 0, 0)),
        out_shape=jax.ShapeDtypeStruct((_B, _OUT, _OUT, _TOTAL_C), jnp.float32),
        scratch_shapes=[pltpu.VMEM((_HB, _OUT, 128), jnp.float32)],
    )(p0, f1, f2, f3)
    return jnp.transpose(out, (0, 3, 1, 2))


# speedup vs baseline: 1.0485x; 1.0485x over previous
import numpy as np
import jax
import jax.numpy as jnp
from jax.experimental import pallas as pl
from jax.experimental.pallas import tpu as pltpu

_OUT = 64
_TOTAL_C = 1440
_B = 4


def _pool_matrix() -> np.ndarray:
    m = np.zeros((128, _OUT), dtype=np.float32)
    idx = np.arange(_OUT)
    m[2 * idx, idx] = 0.25
    m[2 * idx + 1, idx] = 0.25
    return m


_RXT0 = _pool_matrix()


def _pool_body(x_ref, rxt_ref, o_ref):
    s = (x_ref[0:1, :, 0::2, :] + x_ref[0:1, :, 1::2, :])[0]
    y = jnp.dot(s.reshape(96 * _OUT, 128), rxt_ref[...],
                preferred_element_type=jnp.float32)
    o_ref[0] = y.reshape(96, _OUT, _OUT)


def _pool_feat0(feat0):
    return pl.pallas_call(
        _pool_body,
        grid=(_B,),
        in_specs=[pl.BlockSpec((1, 96, 128, 128), lambda b: (b, 0, 0, 0)),
                  pl.BlockSpec((128, _OUT), lambda b: (0, 0))],
        out_specs=pl.BlockSpec((1, 96, _OUT, _OUT), lambda b: (b, 0, 0, 0)),
        out_shape=jax.ShapeDtypeStruct((_B, 96, _OUT, _OUT), jnp.float32),
    )(feat0, _RXT0)


def _upsample_half(x, xd, xu, scale, weights, ch_base, o_ref, scr_ref):
    half = scale // 2
    c = x.shape[-1]
    for k in range(c // 128):
        lo = 128 * k
        xc, xdc, xuc = x[:, :, lo:lo + 128], xd[:, :, lo:lo + 128], xu[:, :, lo:lo + 128]
        rows = [w * xdc + (1.0 - w) * xc for w in weights[:half]]
        rows += [w * xuc + (1.0 - w) * xc for w in weights[half:]]
        for p, r in enumerate(rows):
            rd = jnp.concatenate([r[:, :1], r[:, :-1]], axis=1)
            ru = jnp.concatenate([r[:, 1:], r[:, -1:]], axis=1)
            cols = [w * rd + (1.0 - w) * r for w in weights[:half]]
            cols += [w * ru + (1.0 - w) * r for w in weights[half:]]
            for q, v in enumerate(cols):
                if scale == 2:
                    scr_ref[p::2, q::2, :] = v
                else:
                    scr_ref[p::4, q::4, :] = v
        o_ref[0:1, :, :, ch_base + lo:ch_base + lo + 128] = scr_ref[...][None]


def _body(p0_ref, f1_ref, f2_ref, f3_ref, o_ref, scr_ref):
    h = pl.program_id(1)

    o_ref[0:1, :, :, 0:96] = p0_ref[0:1]
    o_ref[0:1, :, :, 96:288] = f1_ref[0:1]

    def level(x_ref, scale, weights, ch_base):
        n = x_ref.shape[1]
        m = n // 2

        @pl.when(h == 0)
        def _():
            x = x_ref[0, 0:m]
            xd = jnp.concatenate([x[:1], x[:-1]], axis=0)
            xu = x_ref[0, 1:m + 1]
            _upsample_half(x, xd, xu, scale, weights, ch_base, o_ref, scr_ref)

        @pl.when(h == 1)
        def _():
            x = x_ref[0, m:n]
            xd = x_ref[0, m - 1:n - 1]
            xu = jnp.concatenate([x[1:], x[-1:]], axis=0)
            _upsample_half(x, xd, xu, scale, weights, ch_base, o_ref, scr_ref)

    level(f2_ref, 2, (0.25, 0.25), 288)
    level(f3_ref, 4, (0.375, 0.125, 0.125, 0.375), 672)


def kernel(feat0, feat1, feat2, feat3):
    p0 = jnp.transpose(_pool_feat0(feat0), (0, 2, 3, 1))
    f1 = jnp.transpose(feat1, (0, 2, 3, 1))
    f2 = jnp.transpose(feat2, (0, 2, 3, 1))
    f3 = jnp.transpose(feat3, (0, 2, 3, 1))
    out = pl.pallas_call(
        _body,
        grid=(_B, 2),
        in_specs=[pl.BlockSpec((1, 32, _OUT, 96), lambda b, h: (b, h, 0, 0)),
                  pl.BlockSpec((1, 32, _OUT, 192), lambda b, h: (b, h, 0, 0)),
                  pl.BlockSpec((1, 32, 32, 384), lambda b, h: (b, 0, 0, 0)),
                  pl.BlockSpec((1, 16, 16, 768), lambda b, h: (b, 0, 0, 0))],
        out_specs=pl.BlockSpec((1, 32, _OUT, _TOTAL_C),
                               lambda b, h: (b, h, 0, 0)),
        out_shape=jax.ShapeDtypeStruct((_B, _OUT, _OUT, _TOTAL_C), jnp.float32),
        scratch_shapes=[pltpu.VMEM((32, _OUT, 128), jnp.float32)],
    )(p0, f1, f2, f3)
    return jnp.transpose(out, (0, 3, 1, 2))
